# Initial kernel scaffold; baseline (speedup 1.0000x reference)
import jax
import jax.numpy as jnp
from jax.experimental import pallas as pl


def kernel(x, edge_index, batch, gin_W1, gin_b1, gin_gamma, gin_beta, gin_W2, gin_b2, pA0, pb0, pB0, pc0, pA, pb, pB, pc):
    raise NotImplementedError("write your pallas kernel here")


# speedup vs baseline: 6.4600x; 6.4600x over previous
import functools

import jax
import jax.numpy as jnp
from jax import lax
from jax.experimental import pallas as pl
from jax.experimental.pallas import tpu as pltpu
from jax.experimental.pallas import tpu_sc as plsc

N = 10000
E = 320000
D = 128
H = 128
B = 100
L = 4
POLICY_L = 3

NC = 2
NS = 16
NW = NC * NS
CH = 128
N_CH = -(-E // (NW * CH))
E_PAD = NW * CH * N_CH
ROWS_PT = 632
N_PAD = ROWS_PT * NS
CW = 128

def _seg_sum_body(h_hbm, src_hbm, dst_hbm, zrow_hbm, out_hbm,
                  src_v, dst_v, rows_v, acc_sh, sem):
    c = lax.axis_index("c")
    s = lax.axis_index("s")
    w = c * NS + s
    rs = s * ROWS_PT
    pltpu.sync_copy(zrow_hbm.at[pl.ds(rs, ROWS_PT)],
                    acc_sh.at[pl.ds(rs, ROWS_PT)])
    pltpu.sync_copy(src_hbm.at[w], src_v)
    pltpu.sync_copy(dst_hbm.at[w], dst_v)
    plsc.subcore_barrier()

    def chunk(j, carry):
        pltpu.async_copy(h_hbm.at[src_v.at[j]], rows_v, sem).wait()
        pltpu.sync_copy(rows_v, acc_sh.at[dst_v.at[j]], add=True)
        return carry

    lax.fori_loop(0, N_CH, chunk, 0)
    plsc.subcore_barrier()
    pltpu.sync_copy(acc_sh.at[pl.ds(rs, ROWS_PT)],
                    out_hbm.at[c].at[pl.ds(rs, ROWS_PT)])


@functools.lru_cache(maxsize=None)
def _seg_sum_sc():
    return pl.kernel(
        _seg_sum_body,
        mesh=plsc.VectorSubcoreMesh(core_axis_name="c", subcore_axis_name="s"),
        out_type=jax.ShapeDtypeStruct((NC, N_PAD, D), jnp.float32),
        scratch_types=[
            pltpu.VMEM((N_CH, CH), jnp.int32),
            pltpu.VMEM((N_CH, CH), jnp.int32),
            pltpu.VMEM((CH, D), jnp.float32),
            pltpu.VMEM_SHARED((N_PAD, D), jnp.float32),
            pltpu.SemaphoreType.DMA,
        ],
    )


def _counts_body(dst_hbm, zcnt_hbm, ones_hbm, out_hbm, dst_v, ones_v, cnt_sh):
    c = lax.axis_index("c")
    s = lax.axis_index("s")
    w = c * NS + s
    rs = s * ROWS_PT
    pltpu.sync_copy(zcnt_hbm.at[pl.ds(rs, ROWS_PT)],
                    cnt_sh.at[pl.ds(rs, ROWS_PT)])
    pltpu.sync_copy(ones_hbm, ones_v)
    pltpu.sync_copy(dst_hbm.at[w], dst_v)
    plsc.subcore_barrier()

    def chunk(j, carry):
        pltpu.sync_copy(ones_v, cnt_sh.at[dst_v.at[j]], add=True)
        return carry

    lax.fori_loop(0, N_CH, chunk, 0)
    plsc.subcore_barrier()
    pltpu.sync_copy(cnt_sh.at[pl.ds(rs, ROWS_PT)],
                    out_hbm.at[c].at[pl.ds(rs, ROWS_PT)])


@functools.lru_cache(maxsize=None)
def _counts_sc():
    return pl.kernel(
        _counts_body,
        mesh=plsc.VectorSubcoreMesh(core_axis_name="c", subcore_axis_name="s"),
        out_type=jax.ShapeDtypeStruct((NC, N_PAD, CW), jnp.float32),
        scratch_types=[
            pltpu.VMEM((N_CH, CH), jnp.int32),
            pltpu.VMEM((CH, CW), jnp.float32),
            pltpu.VMEM_SHARED((N_PAD, CW), jnp.float32),
        ],
    )


def _tc_layer_body(h_ref, p_ref, c_ref, w1_ref, b1_ref, g_ref, be_ref,
                   w2_ref, b2_ref, out_ref):
    h = h_ref[...]
    ssum = p_ref[0, :N, :] + p_ref[1, :N, :] + h
    cnt = c_ref[0, :N, 0:1] + c_ref[1, :N, 0:1] + 1.0
    z = h + ssum / cnt
    z1 = jnp.dot(z, w1_ref[...], preferred_element_type=jnp.float32) + b1_ref[...]
    mu = jnp.mean(z1, axis=0, keepdims=True)
    var = jnp.mean((z1 - mu) * (z1 - mu), axis=0, keepdims=True)
    zn = (z1 - mu) * lax.rsqrt(var + 1e-5) * g_ref[...] + be_ref[...]
    zn = jnp.maximum(zn, 0.0)
    out_ref[...] = (jnp.dot(zn, w2_ref[...],
                            preferred_element_type=jnp.float32) + b2_ref[...])


_tc_layer = pl.pallas_call(
    _tc_layer_body,
    out_shape=jax.ShapeDtypeStruct((N, H), jnp.float32),
)


def _tc_head_body(h1_ref, h2_ref, h3_ref, h4_ref, bt_ref,
                  a0n_ref, a0g_ref, b0_ref, bb0_ref, c0_ref,
                  pa_ref, pb_ref, pbb_ref, pc_ref, out_ref):
    f32 = jnp.float32
    npool = h1_ref[...] + h2_ref[...] + h3_ref[...] + h4_ref[...]
    bt = bt_ref[...]
    iota_b = lax.broadcasted_iota(jnp.int32, (B, N), 0)
    eq = (bt == iota_b).astype(f32)
    bcnt = jnp.sum(eq, axis=1, keepdims=True)
    mnorm = eq / jnp.maximum(bcnt, 1.0)
    gp = jnp.dot(mnorm, npool, preferred_element_type=f32)
    row = lax.broadcasted_iota(jnp.int32, (N, B), 0)
    col = lax.broadcasted_iota(jnp.int32, (N, B), 1) * (N // B)
    rep = (jnp.logical_and(row >= col, row < col + (N // B))).astype(f32)
    grph = jnp.dot(rep, gp, preferred_element_type=f32)
    z = jnp.tanh(jnp.dot(npool, a0n_ref[...], preferred_element_type=f32)
                 + jnp.dot(grph, a0g_ref[...], preferred_element_type=f32)
                 + b0_ref[...])
    z = jnp.dot(z, bb0_ref[...], preferred_element_type=f32) + c0_ref[...]
    for l in range(POLICY_L - 1):
        z = jnp.tanh(jnp.dot(z, pa_ref[l], preferred_element_type=f32)
                     + pb_ref[l:l + 1, :])
        z = jnp.dot(z, pbb_ref[l], preferred_element_type=f32) + pc_ref[l:l + 1, :]
    z3 = z.reshape(B, N // B, H)
    out_ref[...] = lax.dot_general(
        z3, z3, (((2,), (2,)), ((0,), (0,))), preferred_element_type=f32)


_tc_head = pl.pallas_call(
    _tc_head_body,
    out_shape=jax.ShapeDtypeStruct((B, N // B, N // B), jnp.float32),
)


def kernel(x, edge_index, batch, gin_W1, gin_b1, gin_gamma, gin_beta,
           gin_W2, gin_b2, pA0, pb0, pB0, pc0, pA, pb, pB, pc):
    src = edge_index[0]
    dst = edge_index[1]
    pad = E_PAD - E
    src_p = jnp.concatenate(
        [src, jnp.zeros((pad,), jnp.int32)]).reshape(NW, N_CH, CH)
    dst_p = jnp.concatenate(
        [dst, jnp.full((pad,), N, jnp.int32)]).reshape(NW, N_CH, CH)
    zrow = jnp.zeros((N_PAD, D), jnp.float32)
    zcnt = jnp.zeros((N_PAD, CW), jnp.float32)
    ones = jnp.ones((CH, CW), jnp.float32)

    cnts = _counts_sc()(dst_p, zcnt, ones)
    hs = []
    h = x
    for l in range(L):
        parts = _seg_sum_sc()(h, src_p, dst_p, zrow)
        h = _tc_layer(h, parts, cnts, gin_W1[l],
                      gin_b1[l].reshape(1, H), gin_gamma[l].reshape(1, H),
                      gin_beta[l].reshape(1, H), gin_W2[l],
                      gin_b2[l].reshape(1, H))
        hs.append(h)

    return _tc_head(hs[0], hs[1], hs[2], hs[3],
                    batch.reshape(1, N).astype(jnp.int32),
                    pA0[:H], pA0[H:], pb0.reshape(1, H), pB0,
                    pc0.reshape(1, H), pA, pb, pB, pc)


# speedup vs baseline: 6.6832x; 1.0346x over previous
import functools

import jax
import jax.numpy as jnp
from jax import lax
from jax.experimental import pallas as pl
from jax.experimental.pallas import tpu as pltpu
from jax.experimental.pallas import tpu_sc as plsc

N = 10000
E = 320000
D = 128
H = 128
B = 100
L = 4
POLICY_L = 3

NC = 2
NS = 16
NW = NC * NS
CH = 128
N_CH = -(-E // (NW * CH))
E_PAD = NW * CH * N_CH
ROWS_PT = 632
N_PAD = ROWS_PT * NS
CW = 128

def _seg_sum_body(h_hbm, srcf_hbm, dst_hbm, zrow_hbm, out_hbm,
                  idx0, idx1, dst_v, rows0, rows1, acc_sh,
                  sem_i0, sem_i1, sem_g, sem_s0, sem_s1):
    c = lax.axis_index("c")
    s = lax.axis_index("s")
    w = c * NS + s
    rs = s * ROWS_PT
    ebase = w * (N_CH * CH)
    pltpu.sync_copy(zrow_hbm.at[pl.ds(rs, ROWS_PT)],
                    acc_sh.at[pl.ds(rs, ROWS_PT)])
    pltpu.sync_copy(dst_hbm.at[w], dst_v)
    plsc.subcore_barrier()

    idx = (idx0, idx1)
    rows = (rows0, rows1)
    sem_i = (sem_i0, sem_i1)
    sem_s = (sem_s0, sem_s1)

    pltpu.make_async_copy(srcf_hbm.at[pl.ds(ebase, CH)], idx0, sem_i0).start()

    def step(j, b):
        nb = 1 - b

        @pl.when(j + 1 < N_CH)
        def _():
            pltpu.make_async_copy(
                srcf_hbm.at[pl.ds(ebase + (j + 1) * CH, CH)],
                idx[nb], sem_i[nb]).start()

        pltpu.make_async_copy(
            srcf_hbm.at[pl.ds(ebase + j * CH, CH)], idx[b], sem_i[b]).wait()

        @pl.when(j >= 2)
        def _():
            pltpu.make_async_copy(
                rows[b], acc_sh.at[dst_v.at[j - 2]], sem_s[b]).wait()

        pltpu.async_copy(h_hbm.at[idx[b]], rows[b], sem_g).wait()
        pltpu.make_async_copy(
            rows[b], acc_sh.at[dst_v.at[j]], sem_s[b]).start(add=True)

    def pair(jj, carry):
        step(jj * 2, 0)
        step(jj * 2 + 1, 1)
        return carry

    lax.fori_loop(0, N_CH // 2, pair, 0)
    if N_CH % 2:
        step(N_CH - 1, 0)
    pltpu.make_async_copy(rows0, acc_sh.at[dst_v.at[N_CH - 1]], sem_s0).wait()
    pltpu.make_async_copy(rows1, acc_sh.at[dst_v.at[N_CH - 2]], sem_s1).wait()
    plsc.subcore_barrier()
    pltpu.sync_copy(acc_sh.at[pl.ds(rs, ROWS_PT)],
                    out_hbm.at[c].at[pl.ds(rs, ROWS_PT)])


@functools.lru_cache(maxsize=None)
def _seg_sum_sc():
    return pl.kernel(
        _seg_sum_body,
        mesh=plsc.VectorSubcoreMesh(core_axis_name="c", subcore_axis_name="s"),
        out_type=jax.ShapeDtypeStruct((NC, N_PAD, D), jnp.float32),
        scratch_types=[
            pltpu.VMEM((CH,), jnp.int32),
            pltpu.VMEM((CH,), jnp.int32),
            pltpu.VMEM((N_CH, CH), jnp.int32),
            pltpu.VMEM((CH, D), jnp.float32),
            pltpu.VMEM((CH, D), jnp.float32),
            pltpu.VMEM_SHARED((N_PAD, D), jnp.float32),
            pltpu.SemaphoreType.DMA,
            pltpu.SemaphoreType.DMA,
            pltpu.SemaphoreType.DMA,
            pltpu.SemaphoreType.DMA,
            pltpu.SemaphoreType.DMA,
        ],
    )


def _counts_body(dst_hbm, zcnt_hbm, ones_hbm, out_hbm, dst_v, ones_v, cnt_sh):
    c = lax.axis_index("c")
    s = lax.axis_index("s")
    w = c * NS + s
    rs = s * ROWS_PT
    pltpu.sync_copy(zcnt_hbm.at[pl.ds(rs, ROWS_PT)],
                    cnt_sh.at[pl.ds(rs, ROWS_PT)])
    pltpu.sync_copy(ones_hbm, ones_v)
    pltpu.sync_copy(dst_hbm.at[w], dst_v)
    plsc.subcore_barrier()

    def chunk(j, carry):
        pltpu.sync_copy(ones_v, cnt_sh.at[dst_v.at[j]], add=True)
        return carry

    lax.fori_loop(0, N_CH, chunk, 0)
    plsc.subcore_barrier()
    pltpu.sync_copy(cnt_sh.at[pl.ds(rs, ROWS_PT)],
                    out_hbm.at[c].at[pl.ds(rs, ROWS_PT)])


@functools.lru_cache(maxsize=None)
def _counts_sc():
    return pl.kernel(
        _counts_body,
        mesh=plsc.VectorSubcoreMesh(core_axis_name="c", subcore_axis_name="s"),
        out_type=jax.ShapeDtypeStruct((NC, N_PAD, CW), jnp.float32),
        scratch_types=[
            pltpu.VMEM((N_CH, CH), jnp.int32),
            pltpu.VMEM((CH, CW), jnp.float32),
            pltpu.VMEM_SHARED((N_PAD, CW), jnp.float32),
        ],
    )


def _tc_layer_body(h_ref, p_ref, c_ref, w1_ref, b1_ref, g_ref, be_ref,
                   w2_ref, b2_ref, out_ref):
    h = h_ref[...]
    ssum = p_ref[0, :N, :] + p_ref[1, :N, :] + h
    cnt = c_ref[0, :N, 0:1] + c_ref[1, :N, 0:1] + 1.0
    z = h + ssum / cnt
    z1 = jnp.dot(z, w1_ref[...], preferred_element_type=jnp.float32) + b1_ref[...]
    mu = jnp.mean(z1, axis=0, keepdims=True)
    var = jnp.mean((z1 - mu) * (z1 - mu), axis=0, keepdims=True)
    zn = (z1 - mu) * lax.rsqrt(var + 1e-5) * g_ref[...] + be_ref[...]
    zn = jnp.maximum(zn, 0.0)
    out_ref[...] = (jnp.dot(zn, w2_ref[...],
                            preferred_element_type=jnp.float32) + b2_ref[...])


_tc_layer = pl.pallas_call(
    _tc_layer_body,
    out_shape=jax.ShapeDtypeStruct((N, H), jnp.float32),
)


def _tc_head_body(h1_ref, h2_ref, h3_ref, h4_ref, bt_ref,
                  a0n_ref, a0g_ref, b0_ref, bb0_ref, c0_ref,
                  pa_ref, pb_ref, pbb_ref, pc_ref, out_ref):
    f32 = jnp.float32
    npool = h1_ref[...] + h2_ref[...] + h3_ref[...] + h4_ref[...]
    bt = bt_ref[...]
    iota_b = lax.broadcasted_iota(jnp.int32, (B, N), 0)
    eq = (bt == iota_b).astype(f32)
    bcnt = jnp.sum(eq, axis=1, keepdims=True)
    mnorm = eq / jnp.maximum(bcnt, 1.0)
    gp = jnp.dot(mnorm, npool, preferred_element_type=f32)
    row = lax.broadcasted_iota(jnp.int32, (N, B), 0)
    col = lax.broadcasted_iota(jnp.int32, (N, B), 1) * (N // B)
    rep = (jnp.logical_and(row >= col, row < col + (N // B))).astype(f32)
    grph = jnp.dot(rep, gp, preferred_element_type=f32)
    z = jnp.tanh(jnp.dot(npool, a0n_ref[...], preferred_element_type=f32)
                 + jnp.dot(grph, a0g_ref[...], preferred_element_type=f32)
                 + b0_ref[...])
    z = jnp.dot(z, bb0_ref[...], preferred_element_type=f32) + c0_ref[...]
    for l in range(POLICY_L - 1):
        z = jnp.tanh(jnp.dot(z, pa_ref[l], preferred_element_type=f32)
                     + pb_ref[l:l + 1, :])
        z = jnp.dot(z, pbb_ref[l], preferred_element_type=f32) + pc_ref[l:l + 1, :]
    z3 = z.reshape(B, N // B, H)
    out_ref[...] = lax.dot_general(
        z3, z3, (((2,), (2,)), ((0,), (0,))), preferred_element_type=f32)


_tc_head = pl.pallas_call(
    _tc_head_body,
    out_shape=jax.ShapeDtypeStruct((B, N // B, N // B), jnp.float32),
)


def kernel(x, edge_index, batch, gin_W1, gin_b1, gin_gamma, gin_beta,
           gin_W2, gin_b2, pA0, pb0, pB0, pc0, pA, pb, pB, pc):
    src = edge_index[0]
    dst = edge_index[1]
    pad = E_PAD - E
    src_p = jnp.concatenate([src, jnp.zeros((pad,), jnp.int32)])
    dst_p = jnp.concatenate(
        [dst, jnp.full((pad,), N, jnp.int32)]).reshape(NW, N_CH, CH)
    zrow = jnp.zeros((N_PAD, D), jnp.float32)
    zcnt = jnp.zeros((N_PAD, CW), jnp.float32)
    ones = jnp.ones((CH, CW), jnp.float32)

    cnts = _counts_sc()(dst_p, zcnt, ones)
    hs = []
    h = x
    for l in range(L):
        parts = _seg_sum_sc()(h, src_p, dst_p, zrow)
        h = _tc_layer(h, parts, cnts, gin_W1[l],
                      gin_b1[l].reshape(1, H), gin_gamma[l].reshape(1, H),
                      gin_beta[l].reshape(1, H), gin_W2[l],
                      gin_b2[l].reshape(1, H))
        hs.append(h)

    return _tc_head(hs[0], hs[1], hs[2], hs[3],
                    batch.reshape(1, N).astype(jnp.int32),
                    pA0[:H], pA0[H:], pb0.reshape(1, H), pB0,
                    pc0.reshape(1, H), pA, pb, pB, pc)


# speedup vs baseline: 12.5577x; 1.8790x over previous
import functools

import jax
import jax.numpy as jnp
from jax import lax
from jax.experimental import pallas as pl
from jax.experimental.pallas import tpu as pltpu
from jax.experimental.pallas import tpu_sc as plsc

N = 10000
E = 320000
D = 128
H = 128
B = 100
L = 4
POLICY_L = 3

NC = 2
NS = 16
NW = NC * NS
CH = 128
N_CH = -(-E // (NW * CH))
E_PAD = NW * CH * N_CH
SCH = 88
SN_CH = -(-E // (NW * SCH))
SE_PAD = NW * SCH * SN_CH
ROWS_PT = 632
N_PAD = ROWS_PT * NS
CW = 128

_NB = 4


def _seg_sum_body(h_hbm, srcf_hbm, dstf_hbm, zrow_hbm, out_hbm, *rest):
    si = rest[0:_NB]
    di = rest[_NB:2 * _NB]
    rows = rest[2 * _NB:3 * _NB]
    acc_sh = rest[3 * _NB]
    sem_i = rest[3 * _NB + 1:4 * _NB + 1]
    sem_g = rest[4 * _NB + 1:5 * _NB + 1]
    sem_s = rest[5 * _NB + 1:6 * _NB + 1]
    c = lax.axis_index("c")
    s = lax.axis_index("s")
    w = c * NS + s
    rs = s * ROWS_PT
    ebase = w * (SN_CH * SCH)
    pltpu.sync_copy(zrow_hbm.at[pl.ds(rs, ROWS_PT)],
                    acc_sh.at[pl.ds(rs, ROWS_PT)])
    plsc.subcore_barrier()

    def idx_start(j, b):
        pltpu.make_async_copy(
            srcf_hbm.at[pl.ds(ebase + j * SCH, SCH)], si[b], sem_i[b]).start()
        pltpu.make_async_copy(
            dstf_hbm.at[pl.ds(ebase + j * SCH, SCH)], di[b], sem_i[b]).start()

    def idx_wait(j, b):
        pltpu.make_async_copy(
            srcf_hbm.at[pl.ds(ebase + j * SCH, SCH)], si[b], sem_i[b]).wait()
        pltpu.make_async_copy(
            dstf_hbm.at[pl.ds(ebase + j * SCH, SCH)], di[b], sem_i[b]).wait()

    def gather_start(b):
        pltpu.make_async_copy(h_hbm.at[si[b]], rows[b], sem_g[b]).start()

    def gather_wait(b):
        pltpu.make_async_copy(h_hbm.at[si[b]], rows[b], sem_g[b]).wait()

    def scat_start(b):
        pltpu.make_async_copy(
            rows[b], acc_sh.at[di[b]], sem_s[b]).start(add=True)

    def scat_wait(b):
        pltpu.make_async_copy(rows[b], acc_sh.at[di[b]], sem_s[b]).wait()

    idx_start(0, 0)
    idx_start(1, 1)
    idx_wait(0, 0)
    gather_start(0)
    idx_wait(1, 1)
    gather_start(1)

    def step(j, b):
        nb = (b + 2) % _NB
        j_ = jnp.asarray(j)
        more = j_ + 2 < SN_CH

        @pl.when(jnp.logical_and(j_ >= 2, more))
        def _():
            scat_wait(nb)

        @pl.when(more)
        def _():
            idx_start(j + 2, nb)
        gather_wait(b)
        scat_start(b)

        @pl.when(more)
        def _():
            idx_wait(j + 2, nb)
            gather_start(nb)

    def quad(jj, carry):
        step(jj * 4, 0)
        step(jj * 4 + 1, 1)
        step(jj * 4 + 2, 2)
        step(jj * 4 + 3, 3)
        return carry

    lax.fori_loop(0, SN_CH // 4, quad, 0)
    for r in range(SN_CH - SN_CH % 4, SN_CH):
        step(r, r % _NB)
    for b in range(_NB):
        scat_wait(b)
    plsc.subcore_barrier()
    pltpu.sync_copy(acc_sh.at[pl.ds(rs, ROWS_PT)],
                    out_hbm.at[c].at[pl.ds(rs, ROWS_PT)])


@functools.lru_cache(maxsize=None)
def _seg_sum_sc():
    return pl.kernel(
        _seg_sum_body,
        mesh=plsc.VectorSubcoreMesh(core_axis_name="c", subcore_axis_name="s"),
        out_type=jax.ShapeDtypeStruct((NC, N_PAD, D), jnp.float32),
        scratch_types=(
            [pltpu.VMEM((SCH,), jnp.int32) for _ in range(_NB)]
            + [pltpu.VMEM((SCH,), jnp.int32) for _ in range(_NB)]
            + [pltpu.VMEM((SCH, D), jnp.float32) for _ in range(_NB)]
            + [pltpu.VMEM_SHARED((N_PAD, D), jnp.float32)]
            + [pltpu.SemaphoreType.DMA for _ in range(3 * _NB)]
        ),
    )


def _counts_body(dst_hbm, zcnt_hbm, ones_hbm, out_hbm, dst_v, ones_v, cnt_sh):
    c = lax.axis_index("c")
    s = lax.axis_index("s")
    w = c * NS + s
    rs = s * ROWS_PT
    pltpu.sync_copy(zcnt_hbm.at[pl.ds(rs, ROWS_PT)],
                    cnt_sh.at[pl.ds(rs, ROWS_PT)])
    pltpu.sync_copy(ones_hbm, ones_v)
    pltpu.sync_copy(dst_hbm.at[w], dst_v)
    plsc.subcore_barrier()

    def chunk(j, carry):
        pltpu.sync_copy(ones_v, cnt_sh.at[dst_v.at[j]], add=True)
        return carry

    lax.fori_loop(0, N_CH, chunk, 0)
    plsc.subcore_barrier()
    pltpu.sync_copy(cnt_sh.at[pl.ds(rs, ROWS_PT)],
                    out_hbm.at[c].at[pl.ds(rs, ROWS_PT)])


@functools.lru_cache(maxsize=None)
def _counts_sc():
    return pl.kernel(
        _counts_body,
        mesh=plsc.VectorSubcoreMesh(core_axis_name="c", subcore_axis_name="s"),
        out_type=jax.ShapeDtypeStruct((NC, N_PAD, CW), jnp.float32),
        scratch_types=[
            pltpu.VMEM((N_CH, CH), jnp.int32),
            pltpu.VMEM((CH, CW), jnp.float32),
            pltpu.VMEM_SHARED((N_PAD, CW), jnp.float32),
        ],
    )


def _tc_layer_body(h_ref, p_ref, c_ref, w1_ref, b1_ref, g_ref, be_ref,
                   w2_ref, b2_ref, out_ref):
    h = h_ref[...]
    ssum = p_ref[0, :N, :] + p_ref[1, :N, :] + h
    cnt = c_ref[0, :N, 0:1] + c_ref[1, :N, 0:1] + 1.0
    z = h + ssum / cnt
    z1 = jnp.dot(z, w1_ref[...], preferred_element_type=jnp.float32) + b1_ref[...]
    mu = jnp.mean(z1, axis=0, keepdims=True)
    var = jnp.mean((z1 - mu) * (z1 - mu), axis=0, keepdims=True)
    zn = (z1 - mu) * lax.rsqrt(var + 1e-5) * g_ref[...] + be_ref[...]
    zn = jnp.maximum(zn, 0.0)
    out_ref[...] = (jnp.dot(zn, w2_ref[...],
                            preferred_element_type=jnp.float32) + b2_ref[...])


_tc_layer = pl.pallas_call(
    _tc_layer_body,
    out_shape=jax.ShapeDtypeStruct((N, H), jnp.float32),
)


def _tc_head_body(h1_ref, h2_ref, h3_ref, h4_ref, bt_ref,
                  a0n_ref, a0g_ref, b0_ref, bb0_ref, c0_ref,
                  pa_ref, pb_ref, pbb_ref, pc_ref, out_ref):
    f32 = jnp.float32
    npool = h1_ref[...] + h2_ref[...] + h3_ref[...] + h4_ref[...]
    bt = bt_ref[...]
    iota_b = lax.broadcasted_iota(jnp.int32, (B, N), 0)
    eq = (bt == iota_b).astype(f32)
    bcnt = jnp.sum(eq, axis=1, keepdims=True)
    mnorm = eq / jnp.maximum(bcnt, 1.0)
    gp = jnp.dot(mnorm, npool, preferred_element_type=f32)
    row = lax.broadcasted_iota(jnp.int32, (N, B), 0)
    col = lax.broadcasted_iota(jnp.int32, (N, B), 1) * (N // B)
    rep = (jnp.logical_and(row >= col, row < col + (N // B))).astype(f32)
    grph = jnp.dot(rep, gp, preferred_element_type=f32)
    z = jnp.tanh(jnp.dot(npool, a0n_ref[...], preferred_element_type=f32)
                 + jnp.dot(grph, a0g_ref[...], preferred_element_type=f32)
                 + b0_ref[...])
    z = jnp.dot(z, bb0_ref[...], preferred_element_type=f32) + c0_ref[...]
    for l in range(POLICY_L - 1):
        z = jnp.tanh(jnp.dot(z, pa_ref[l], preferred_element_type=f32)
                     + pb_ref[l:l + 1, :])
        z = jnp.dot(z, pbb_ref[l], preferred_element_type=f32) + pc_ref[l:l + 1, :]
    z3 = z.reshape(B, N // B, H)
    out_ref[...] = lax.dot_general(
        z3, z3, (((2,), (2,)), ((0,), (0,))), preferred_element_type=f32)


_tc_head = pl.pallas_call(
    _tc_head_body,
    out_shape=jax.ShapeDtypeStruct((B, N // B, N // B), jnp.float32),
)


def kernel(x, edge_index, batch, gin_W1, gin_b1, gin_gamma, gin_beta,
           gin_W2, gin_b2, pA0, pb0, pB0, pc0, pA, pb, pB, pc):
    src = edge_index[0]
    dst = edge_index[1]
    spad = SE_PAD - E
    src_f = jnp.concatenate([src, jnp.zeros((spad,), jnp.int32)])
    dst_f = jnp.concatenate([dst, jnp.full((spad,), N, jnp.int32)])
    pad = E_PAD - E
    dst_p = jnp.concatenate(
        [dst, jnp.full((pad,), N, jnp.int32)]).reshape(NW, N_CH, CH)
    zrow = jnp.zeros((N_PAD, D), jnp.float32)
    zcnt = jnp.zeros((N_PAD, CW), jnp.float32)
    ones = jnp.ones((CH, CW), jnp.float32)

    cnts = _counts_sc()(dst_p, zcnt, ones)
    hs = []
    h = x
    for l in range(L):
        parts = _seg_sum_sc()(h, src_f, dst_f, zrow)
        h = _tc_layer(h, parts, cnts, gin_W1[l],
                      gin_b1[l].reshape(1, H), gin_gamma[l].reshape(1, H),
                      gin_beta[l].reshape(1, H), gin_W2[l],
                      gin_b2[l].reshape(1, H))
        hs.append(h)

    return _tc_head(hs[0], hs[1], hs[2], hs[3],
                    batch.reshape(1, N).astype(jnp.int32),
                    pA0[:H], pA0[H:], pb0.reshape(1, H), pB0,
                    pc0.reshape(1, H), pA, pb, pB, pc)


# speedup vs baseline: 12.9343x; 1.0300x over previous
import functools

import jax
import jax.numpy as jnp
from jax import lax
from jax.experimental import pallas as pl
from jax.experimental.pallas import tpu as pltpu
from jax.experimental.pallas import tpu_sc as plsc

N = 10000
E = 320000
D = 128
H = 128
B = 100
L = 4
POLICY_L = 3

NC = 2
NS = 16
NW = NC * NS
CH = 128
N_CH = -(-E // (NW * CH))
E_PAD = NW * CH * N_CH
SCH = 56
SN_CH = -(-E // (NW * SCH))
SE_PAD = NW * SCH * SN_CH
ROWS_PT = 632
N_PAD = ROWS_PT * NS
CW = 128

_NB = 6
_G = 3


def _seg_sum_body(h_hbm, srcf_hbm, dstf_hbm, zrow_hbm, out_hbm, *rest):
    si = rest[0:_NB]
    di = rest[_NB:2 * _NB]
    rows = rest[2 * _NB:3 * _NB]
    acc_sh = rest[3 * _NB]
    sem_i = rest[3 * _NB + 1:4 * _NB + 1]
    sem_g = rest[4 * _NB + 1:5 * _NB + 1]
    sem_s = rest[5 * _NB + 1:6 * _NB + 1]
    c = lax.axis_index("c")
    s = lax.axis_index("s")
    w = c * NS + s
    rs = s * ROWS_PT
    ebase = w * (SN_CH * SCH)
    pltpu.sync_copy(zrow_hbm.at[pl.ds(rs, ROWS_PT)],
                    acc_sh.at[pl.ds(rs, ROWS_PT)])
    plsc.subcore_barrier()

    def idx_start(j, b):
        pltpu.make_async_copy(
            srcf_hbm.at[pl.ds(ebase + j * SCH, SCH)], si[b], sem_i[b]).start()
        pltpu.make_async_copy(
            dstf_hbm.at[pl.ds(ebase + j * SCH, SCH)], di[b], sem_i[b]).start()

    def idx_wait(j, b):
        pltpu.make_async_copy(
            srcf_hbm.at[pl.ds(ebase + j * SCH, SCH)], si[b], sem_i[b]).wait()
        pltpu.make_async_copy(
            dstf_hbm.at[pl.ds(ebase + j * SCH, SCH)], di[b], sem_i[b]).wait()

    def gather_start(b):
        pltpu.make_async_copy(h_hbm.at[si[b]], rows[b], sem_g[b]).start()

    def gather_wait(b):
        pltpu.make_async_copy(h_hbm.at[si[b]], rows[b], sem_g[b]).wait()

    def scat_start(b):
        pltpu.make_async_copy(
            rows[b], acc_sh.at[di[b]], sem_s[b]).start(add=True)

    def scat_wait(b):
        pltpu.make_async_copy(rows[b], acc_sh.at[di[b]], sem_s[b]).wait()

    for b0 in range(_G):
        idx_start(b0, b0)
    for b0 in range(_G):
        idx_wait(b0, b0)
        gather_start(b0)

    def step(j, b):
        nb = (b + _G) % _NB
        j_ = jnp.asarray(j)
        more = j_ + _G < SN_CH

        @pl.when(jnp.logical_and(j_ >= _NB - _G, more))
        def _():
            scat_wait(nb)

        @pl.when(more)
        def _():
            idx_start(j + _G, nb)
        gather_wait(b)
        scat_start(b)

        @pl.when(more)
        def _():
            idx_wait(j + _G, nb)
            gather_start(nb)

    def group(jj, carry):
        for k in range(_NB):
            step(jj * _NB + k, k)
        return carry

    lax.fori_loop(0, SN_CH // _NB, group, 0)
    for r in range(SN_CH - SN_CH % _NB, SN_CH):
        step(r, r % _NB)
    for b in range(_NB):
        scat_wait(b)
    plsc.subcore_barrier()
    pltpu.sync_copy(acc_sh.at[pl.ds(rs, ROWS_PT)],
                    out_hbm.at[c].at[pl.ds(rs, ROWS_PT)])


@functools.lru_cache(maxsize=None)
def _seg_sum_sc():
    return pl.kernel(
        _seg_sum_body,
        mesh=plsc.VectorSubcoreMesh(core_axis_name="c", subcore_axis_name="s"),
        out_type=jax.ShapeDtypeStruct((NC, N_PAD, D), jnp.float32),
        scratch_types=(
            [pltpu.VMEM((SCH,), jnp.int32) for _ in range(_NB)]
            + [pltpu.VMEM((SCH,), jnp.int32) for _ in range(_NB)]
            + [pltpu.VMEM((SCH, D), jnp.float32) for _ in range(_NB)]
            + [pltpu.VMEM_SHARED((N_PAD, D), jnp.float32)]
            + [pltpu.SemaphoreType.DMA for _ in range(3 * _NB)]
        ),
    )


def _chk():
    assert SN_CH >= _NB + _G and _G < _NB


_chk()


def _counts_body(dst_hbm, zcnt_hbm, ones_hbm, out_hbm, dst_v, ones_v, cnt_sh):
    c = lax.axis_index("c")
    s = lax.axis_index("s")
    w = c * NS + s
    rs = s * ROWS_PT
    pltpu.sync_copy(zcnt_hbm.at[pl.ds(rs, ROWS_PT)],
                    cnt_sh.at[pl.ds(rs, ROWS_PT)])
    pltpu.sync_copy(ones_hbm, ones_v)
    pltpu.sync_copy(dst_hbm.at[w], dst_v)
    plsc.subcore_barrier()

    def chunk(j, carry):
        pltpu.sync_copy(ones_v, cnt_sh.at[dst_v.at[j]], add=True)
        return carry

    lax.fori_loop(0, N_CH, chunk, 0)
    plsc.subcore_barrier()
    pltpu.sync_copy(cnt_sh.at[pl.ds(rs, ROWS_PT)],
                    out_hbm.at[c].at[pl.ds(rs, ROWS_PT)])


@functools.lru_cache(maxsize=None)
def _counts_sc():
    return pl.kernel(
        _counts_body,
        mesh=plsc.VectorSubcoreMesh(core_axis_name="c", subcore_axis_name="s"),
        out_type=jax.ShapeDtypeStruct((NC, N_PAD, CW), jnp.float32),
        scratch_types=[
            pltpu.VMEM((N_CH, CH), jnp.int32),
            pltpu.VMEM((CH, CW), jnp.float32),
            pltpu.VMEM_SHARED((N_PAD, CW), jnp.float32),
        ],
    )


def _tc_layer_body(h_ref, p_ref, c_ref, w1_ref, b1_ref, g_ref, be_ref,
                   w2_ref, b2_ref, out_ref):
    h = h_ref[...]
    ssum = p_ref[0, :N, :] + p_ref[1, :N, :] + h
    cnt = c_ref[0, :N, 0:1] + c_ref[1, :N, 0:1] + 1.0
    z = h + ssum / cnt
    z1 = jnp.dot(z, w1_ref[...], preferred_element_type=jnp.float32) + b1_ref[...]
    mu = jnp.mean(z1, axis=0, keepdims=True)
    var = jnp.mean((z1 - mu) * (z1 - mu), axis=0, keepdims=True)
    zn = (z1 - mu) * lax.rsqrt(var + 1e-5) * g_ref[...] + be_ref[...]
    zn = jnp.maximum(zn, 0.0)
    out_ref[...] = (jnp.dot(zn, w2_ref[...],
                            preferred_element_type=jnp.float32) + b2_ref[...])


_tc_layer = pl.pallas_call(
    _tc_layer_body,
    out_shape=jax.ShapeDtypeStruct((N, H), jnp.float32),
)


def _tc_head_body(h1_ref, h2_ref, h3_ref, h4_ref, bt_ref,
                  a0n_ref, a0g_ref, b0_ref, bb0_ref, c0_ref,
                  pa_ref, pb_ref, pbb_ref, pc_ref, out_ref):
    f32 = jnp.float32
    npool = h1_ref[...] + h2_ref[...] + h3_ref[...] + h4_ref[...]
    bt = bt_ref[...]
    iota_b = lax.broadcasted_iota(jnp.int32, (B, N), 0)
    eq = (bt == iota_b).astype(f32)
    bcnt = jnp.sum(eq, axis=1, keepdims=True)
    mnorm = eq / jnp.maximum(bcnt, 1.0)
    gp = jnp.dot(mnorm, npool, preferred_element_type=f32)
    row = lax.broadcasted_iota(jnp.int32, (N, B), 0)
    col = lax.broadcasted_iota(jnp.int32, (N, B), 1) * (N // B)
    rep = (jnp.logical_and(row >= col, row < col + (N // B))).astype(f32)
    grph = jnp.dot(rep, gp, preferred_element_type=f32)
    z = jnp.tanh(jnp.dot(npool, a0n_ref[...], preferred_element_type=f32)
                 + jnp.dot(grph, a0g_ref[...], preferred_element_type=f32)
                 + b0_ref[...])
    z = jnp.dot(z, bb0_ref[...], preferred_element_type=f32) + c0_ref[...]
    for l in range(POLICY_L - 1):
        z = jnp.tanh(jnp.dot(z, pa_ref[l], preferred_element_type=f32)
                     + pb_ref[l:l + 1, :])
        z = jnp.dot(z, pbb_ref[l], preferred_element_type=f32) + pc_ref[l:l + 1, :]
    z3 = z.reshape(B, N // B, H)
    out_ref[...] = lax.dot_general(
        z3, z3, (((2,), (2,)), ((0,), (0,))), preferred_element_type=f32)


_tc_head = pl.pallas_call(
    _tc_head_body,
    out_shape=jax.ShapeDtypeStruct((B, N // B, N // B), jnp.float32),
)


def kernel(x, edge_index, batch, gin_W1, gin_b1, gin_gamma, gin_beta,
           gin_W2, gin_b2, pA0, pb0, pB0, pc0, pA, pb, pB, pc):
    src = edge_index[0]
    dst = edge_index[1]
    spad = SE_PAD - E
    src_f = jnp.concatenate([src, jnp.zeros((spad,), jnp.int32)])
    dst_f = jnp.concatenate([dst, jnp.full((spad,), N, jnp.int32)])
    pad = E_PAD - E
    dst_p = jnp.concatenate(
        [dst, jnp.full((pad,), N, jnp.int32)]).reshape(NW, N_CH, CH)
    zrow = jnp.zeros((N_PAD, D), jnp.float32)
    zcnt = jnp.zeros((N_PAD, CW), jnp.float32)
    ones = jnp.ones((CH, CW), jnp.float32)

    cnts = _counts_sc()(dst_p, zcnt, ones)
    hs = []
    h = x
    for l in range(L):
        parts = _seg_sum_sc()(h, src_f, dst_f, zrow)
        h = _tc_layer(h, parts, cnts, gin_W1[l],
                      gin_b1[l].reshape(1, H), gin_gamma[l].reshape(1, H),
                      gin_beta[l].reshape(1, H), gin_W2[l],
                      gin_b2[l].reshape(1, H))
        hs.append(h)

    return _tc_head(hs[0], hs[1], hs[2], hs[3],
                    batch.reshape(1, N).astype(jnp.int32),
                    pA0[:H], pA0[H:], pb0.reshape(1, H), pB0,
                    pc0.reshape(1, H), pA, pb, pB, pc)


# speedup vs baseline: 12.9502x; 1.0012x over previous
import functools

import jax
import jax.numpy as jnp
from jax import lax
from jax.experimental import pallas as pl
from jax.experimental.pallas import tpu as pltpu
from jax.experimental.pallas import tpu_sc as plsc

N = 10000
E = 320000
D = 128
H = 128
B = 100
L = 4
POLICY_L = 3

NC = 2
NS = 16
NW = NC * NS
CH = 128
N_CH = -(-E // (NW * CH))
E_PAD = NW * CH * N_CH
SCH = 56
SN_CH = -(-E // (NW * SCH))
SE_PAD = NW * SCH * SN_CH
ROWS_PT = 632
N_PAD = ROWS_PT * NS
CW = 128

_NB = 6
_G = 3


def _seg_sum_body(h_hbm, srcf_hbm, dstf_hbm, zrow_hbm, out_hbm, *rest):
    si = rest[0:_NB]
    di = rest[_NB:2 * _NB]
    rows = rest[2 * _NB:3 * _NB]
    acc_sh = rest[3 * _NB]
    sem_i = rest[3 * _NB + 1:4 * _NB + 1]
    sem_g = rest[4 * _NB + 1:5 * _NB + 1]
    sem_s = rest[5 * _NB + 1:6 * _NB + 1]
    c = lax.axis_index("c")
    s = lax.axis_index("s")
    w = c * NS + s
    rs = s * ROWS_PT
    ebase = w * (SN_CH * SCH)
    pltpu.sync_copy(zrow_hbm.at[pl.ds(rs, ROWS_PT)],
                    acc_sh.at[pl.ds(rs, ROWS_PT)])
    plsc.subcore_barrier()

    def idx_start(j, b):
        pltpu.make_async_copy(
            srcf_hbm.at[pl.ds(ebase + j * SCH, SCH)], si[b], sem_i[b]).start()
        pltpu.make_async_copy(
            dstf_hbm.at[pl.ds(ebase + j * SCH, SCH)], di[b], sem_i[b]).start()

    def idx_wait(j, b):
        pltpu.make_async_copy(
            srcf_hbm.at[pl.ds(ebase + j * SCH, SCH)], si[b], sem_i[b]).wait()
        pltpu.make_async_copy(
            dstf_hbm.at[pl.ds(ebase + j * SCH, SCH)], di[b], sem_i[b]).wait()

    def gather_start(b):
        pltpu.make_async_copy(h_hbm.at[si[b]], rows[b], sem_g[b]).start()

    def gather_wait(b):
        pltpu.make_async_copy(h_hbm.at[si[b]], rows[b], sem_g[b]).wait()

    def scat_start(b):
        pltpu.make_async_copy(
            rows[b], acc_sh.at[di[b]], sem_s[b]).start(add=True)

    def scat_wait(b):
        pltpu.make_async_copy(rows[b], acc_sh.at[di[b]], sem_s[b]).wait()

    for b0 in range(_G):
        idx_start(b0, b0)
    for b0 in range(_G):
        idx_wait(b0, b0)
        gather_start(b0)

    def step(j, b):
        nb = (b + _G) % _NB
        j_ = jnp.asarray(j)
        more = j_ + _G < SN_CH

        @pl.when(jnp.logical_and(j_ >= _NB - _G, more))
        def _():
            scat_wait(nb)

        @pl.when(more)
        def _():
            idx_start(j + _G, nb)
        gather_wait(b)
        scat_start(b)

        @pl.when(more)
        def _():
            idx_wait(j + _G, nb)
            gather_start(nb)

    def group(jj, carry):
        for k in range(_NB):
            step(jj * _NB + k, k)
        return carry

    lax.fori_loop(0, SN_CH // _NB, group, 0)
    for r in range(SN_CH - SN_CH % _NB, SN_CH):
        step(r, r % _NB)
    for b in range(_NB):
        scat_wait(b)
    plsc.subcore_barrier()
    pltpu.sync_copy(acc_sh.at[pl.ds(rs, ROWS_PT)],
                    out_hbm.at[c].at[pl.ds(rs, ROWS_PT)])


@functools.lru_cache(maxsize=None)
def _seg_sum_sc():
    return pl.kernel(
        _seg_sum_body,
        mesh=plsc.VectorSubcoreMesh(core_axis_name="c", subcore_axis_name="s"),
        out_type=jax.ShapeDtypeStruct((NC, N_PAD, D), jnp.float32),
        scratch_types=(
            [pltpu.VMEM((SCH,), jnp.int32) for _ in range(_NB)]
            + [pltpu.VMEM((SCH,), jnp.int32) for _ in range(_NB)]
            + [pltpu.VMEM((SCH, D), jnp.float32) for _ in range(_NB)]
            + [pltpu.VMEM_SHARED((N_PAD, D), jnp.float32)]
            + [pltpu.SemaphoreType.DMA for _ in range(3 * _NB)]
        ),
    )


def _chk():
    assert SN_CH >= _NB + _G and _G < _NB


_chk()


def _counts_body(dst_hbm, zcnt_hbm, ones_hbm, out_hbm, dst_v, ones_v, cnt_sh,
                 sem):
    c = lax.axis_index("c")
    s = lax.axis_index("s")
    w = c * NS + s
    rs = s * ROWS_PT
    pltpu.sync_copy(zcnt_hbm.at[pl.ds(rs, ROWS_PT)],
                    cnt_sh.at[pl.ds(rs, ROWS_PT)])
    pltpu.sync_copy(ones_hbm, ones_v)
    pltpu.sync_copy(dst_hbm.at[w], dst_v)
    plsc.subcore_barrier()

    def chunk(j, carry):
        pltpu.make_async_copy(
            ones_v, cnt_sh.at[dst_v.at[j]], sem).start(add=True)
        return carry

    lax.fori_loop(0, N_CH, chunk, 0)

    def drain(j, carry):
        pltpu.make_async_copy(ones_v, cnt_sh.at[dst_v.at[0]], sem).wait()
        return carry

    lax.fori_loop(0, N_CH, drain, 0)
    plsc.subcore_barrier()
    pltpu.sync_copy(cnt_sh.at[pl.ds(rs, ROWS_PT)],
                    out_hbm.at[c].at[pl.ds(rs, ROWS_PT)])


@functools.lru_cache(maxsize=None)
def _counts_sc():
    return pl.kernel(
        _counts_body,
        mesh=plsc.VectorSubcoreMesh(core_axis_name="c", subcore_axis_name="s"),
        out_type=jax.ShapeDtypeStruct((NC, N_PAD, CW), jnp.float32),
        scratch_types=[
            pltpu.VMEM((N_CH, CH), jnp.int32),
            pltpu.VMEM((CH, CW), jnp.float32),
            pltpu.VMEM_SHARED((N_PAD, CW), jnp.float32),
            pltpu.SemaphoreType.DMA,
        ],
    )


def _tc_layer_body(h_ref, p_ref, c_ref, w1_ref, b1_ref, g_ref, be_ref,
                   w2_ref, b2_ref, out_ref):
    h = h_ref[...]
    ssum = p_ref[0, :N, :] + p_ref[1, :N, :] + h
    cnt = c_ref[0, :N, 0:1] + c_ref[1, :N, 0:1] + 1.0
    z = h + ssum / cnt
    z1 = jnp.dot(z, w1_ref[...], preferred_element_type=jnp.float32) + b1_ref[...]
    mu = jnp.mean(z1, axis=0, keepdims=True)
    var = jnp.mean((z1 - mu) * (z1 - mu), axis=0, keepdims=True)
    zn = (z1 - mu) * lax.rsqrt(var + 1e-5) * g_ref[...] + be_ref[...]
    zn = jnp.maximum(zn, 0.0)
    out_ref[...] = (jnp.dot(zn, w2_ref[...],
                            preferred_element_type=jnp.float32) + b2_ref[...])


_tc_layer = pl.pallas_call(
    _tc_layer_body,
    out_shape=jax.ShapeDtypeStruct((N, H), jnp.float32),
)


def _tc_head_body(h1_ref, h2_ref, h3_ref, h4_ref, bt_ref,
                  a0n_ref, a0g_ref, b0_ref, bb0_ref, c0_ref,
                  pa_ref, pb_ref, pbb_ref, pc_ref, out_ref):
    f32 = jnp.float32
    npool = h1_ref[...] + h2_ref[...] + h3_ref[...] + h4_ref[...]
    bt = bt_ref[...]
    iota_b = lax.broadcasted_iota(jnp.int32, (B, N), 0)
    eq = (bt == iota_b).astype(f32)
    bcnt = jnp.sum(eq, axis=1, keepdims=True)
    mnorm = eq / jnp.maximum(bcnt, 1.0)
    gp = jnp.dot(mnorm, npool, preferred_element_type=f32)
    row = lax.broadcasted_iota(jnp.int32, (N, B), 0)
    col = lax.broadcasted_iota(jnp.int32, (N, B), 1) * (N // B)
    rep = (jnp.logical_and(row >= col, row < col + (N // B))).astype(f32)
    grph = jnp.dot(rep, gp, preferred_element_type=f32)
    z = jnp.tanh(jnp.dot(npool, a0n_ref[...], preferred_element_type=f32)
                 + jnp.dot(grph, a0g_ref[...], preferred_element_type=f32)
                 + b0_ref[...])
    z = jnp.dot(z, bb0_ref[...], preferred_element_type=f32) + c0_ref[...]
    for l in range(POLICY_L - 1):
        z = jnp.tanh(jnp.dot(z, pa_ref[l], preferred_element_type=f32)
                     + pb_ref[l:l + 1, :])
        z = jnp.dot(z, pbb_ref[l], preferred_element_type=f32) + pc_ref[l:l + 1, :]
    z3 = z.reshape(B, N // B, H)
    out_ref[...] = lax.dot_general(
        z3, z3, (((2,), (2,)), ((0,), (0,))), preferred_element_type=f32)


_tc_head = pl.pallas_call(
    _tc_head_body,
    out_shape=jax.ShapeDtypeStruct((B, N // B, N // B), jnp.float32),
)


def kernel(x, edge_index, batch, gin_W1, gin_b1, gin_gamma, gin_beta,
           gin_W2, gin_b2, pA0, pb0, pB0, pc0, pA, pb, pB, pc):
    src = edge_index[0]
    dst = edge_index[1]
    spad = SE_PAD - E
    src_f = jnp.concatenate([src, jnp.zeros((spad,), jnp.int32)])
    dst_f = jnp.concatenate([dst, jnp.full((spad,), N, jnp.int32)])
    pad = E_PAD - E
    dst_p = jnp.concatenate(
        [dst, jnp.full((pad,), N, jnp.int32)]).reshape(NW, N_CH, CH)
    zrow = jnp.zeros((N_PAD, D), jnp.float32)
    zcnt = jnp.zeros((N_PAD, CW), jnp.float32)
    ones = jnp.ones((CH, CW), jnp.float32)

    cnts = _counts_sc()(dst_p, zcnt, ones)
    hs = []
    h = x
    for l in range(L):
        parts = _seg_sum_sc()(h, src_f, dst_f, zrow)
        h = _tc_layer(h, parts, cnts, gin_W1[l],
                      gin_b1[l].reshape(1, H), gin_gamma[l].reshape(1, H),
                      gin_beta[l].reshape(1, H), gin_W2[l],
                      gin_b2[l].reshape(1, H))
        hs.append(h)

    return _tc_head(hs[0], hs[1], hs[2], hs[3],
                    batch.reshape(1, N).astype(jnp.int32),
                    pA0[:H], pA0[H:], pb0.reshape(1, H), pB0,
                    pc0.reshape(1, H), pA, pb, pB, pc)


# speedup vs baseline: 13.6268x; 1.0522x over previous
import functools

import jax
import jax.numpy as jnp
from jax import lax
from jax.experimental import pallas as pl
from jax.experimental.pallas import tpu as pltpu
from jax.experimental.pallas import tpu_sc as plsc

N = 10000
E = 320000
D = 128
H = 128
B = 100
L = 4
POLICY_L = 3

NC = 2
NS = 16
NW = NC * NS
CH = 128
N_CH = -(-E // (NW * CH))
E_PAD = NW * CH * N_CH
SCH = 56
SN_CH = -(-E // (NW * SCH))
SN0 = 200
SN1 = 2 * SN_CH - SN0
SE_PAD = NS * SCH * (SN0 + SN1)
E0 = NS * SN0 * SCH
ROWS_PT = 632
N_PAD = ROWS_PT * NS
CW = 128

_NB = 6
_G = 3


def _seg_sum_body(h_hbm, srcf_hbm, dstf_hbm, zrow_hbm, out_hbm, *rest):
    si = rest[0:_NB]
    di = rest[_NB:2 * _NB]
    rows = rest[2 * _NB:3 * _NB]
    acc_sh = rest[3 * _NB]
    sem_i = rest[3 * _NB + 1:4 * _NB + 1]
    sem_g = rest[4 * _NB + 1:5 * _NB + 1]
    sem_s = rest[5 * _NB + 1:6 * _NB + 1]
    c = lax.axis_index("c")
    s = lax.axis_index("s")
    rs = s * ROWS_PT
    pltpu.sync_copy(zrow_hbm.at[pl.ds(rs, ROWS_PT)],
                    acc_sh.at[pl.ds(rs, ROWS_PT)])
    plsc.subcore_barrier()

    def pipeline(ebase, sn_ch):
        def idx_start(j, b):
            pltpu.make_async_copy(
                srcf_hbm.at[pl.ds(ebase + j * SCH, SCH)],
                si[b], sem_i[b]).start()
            pltpu.make_async_copy(
                dstf_hbm.at[pl.ds(ebase + j * SCH, SCH)],
                di[b], sem_i[b]).start()

        def idx_wait(j, b):
            pltpu.make_async_copy(
                srcf_hbm.at[pl.ds(ebase + j * SCH, SCH)],
                si[b], sem_i[b]).wait()
            pltpu.make_async_copy(
                dstf_hbm.at[pl.ds(ebase + j * SCH, SCH)],
                di[b], sem_i[b]).wait()

        def gather_start(b):
            pltpu.make_async_copy(h_hbm.at[si[b]], rows[b], sem_g[b]).start()

        def gather_wait(b):
            pltpu.make_async_copy(h_hbm.at[si[b]], rows[b], sem_g[b]).wait()

        def scat_start(b):
            pltpu.make_async_copy(
                rows[b], acc_sh.at[di[b]], sem_s[b]).start(add=True)

        def scat_wait(b):
            pltpu.make_async_copy(rows[b], acc_sh.at[di[b]], sem_s[b]).wait()

        for b0 in range(_G):
            idx_start(b0, b0)
        for b0 in range(_G):
            idx_wait(b0, b0)
            gather_start(b0)

        def step(j, b):
            nb = (b + _G) % _NB
            j_ = jnp.asarray(j)
            more = j_ + _G < sn_ch

            @pl.when(jnp.logical_and(j_ >= _NB - _G, more))
            def _():
                scat_wait(nb)

            @pl.when(more)
            def _():
                idx_start(j + _G, nb)
            gather_wait(b)
            scat_start(b)

            @pl.when(more)
            def _():
                idx_wait(j + _G, nb)
                gather_start(nb)

        def group(jj, carry):
            for k in range(_NB):
                step(jj * _NB + k, k)
            return carry

        lax.fori_loop(0, sn_ch // _NB, group, 0)
        for r in range(sn_ch - sn_ch % _NB, sn_ch):
            step(r, r % _NB)
        for b in range(_NB):
            scat_wait(b)

    @pl.when(c == 0)
    def _():
        pipeline(s * (SN0 * SCH), SN0)

    @pl.when(c == 1)
    def _():
        pipeline(E0 + s * (SN1 * SCH), SN1)

    plsc.subcore_barrier()
    pltpu.sync_copy(acc_sh.at[pl.ds(rs, ROWS_PT)],
                    out_hbm.at[c].at[pl.ds(rs, ROWS_PT)])


@functools.lru_cache(maxsize=None)
def _seg_sum_sc():
    return pl.kernel(
        _seg_sum_body,
        mesh=plsc.VectorSubcoreMesh(core_axis_name="c", subcore_axis_name="s"),
        out_type=jax.ShapeDtypeStruct((NC, N_PAD, D), jnp.float32),
        scratch_types=(
            [pltpu.VMEM((SCH,), jnp.int32) for _ in range(_NB)]
            + [pltpu.VMEM((SCH,), jnp.int32) for _ in range(_NB)]
            + [pltpu.VMEM((SCH, D), jnp.float32) for _ in range(_NB)]
            + [pltpu.VMEM_SHARED((N_PAD, D), jnp.float32)]
            + [pltpu.SemaphoreType.DMA for _ in range(3 * _NB)]
        ),
    )


def _chk():
    assert min(SN0, SN1) >= _NB + _G and _G < _NB


_chk()


def _counts_body(dst_hbm, zcnt_hbm, ones_hbm, out_hbm, dst_v, ones_v, cnt_sh,
                 sem):
    c = lax.axis_index("c")
    s = lax.axis_index("s")
    w = c * NS + s
    rs = s * ROWS_PT
    pltpu.sync_copy(zcnt_hbm.at[pl.ds(rs, ROWS_PT)],
                    cnt_sh.at[pl.ds(rs, ROWS_PT)])
    pltpu.sync_copy(ones_hbm, ones_v)
    pltpu.sync_copy(dst_hbm.at[w], dst_v)
    plsc.subcore_barrier()

    def chunk(j, carry):
        pltpu.make_async_copy(
            ones_v, cnt_sh.at[dst_v.at[j]], sem).start(add=True)
        return carry

    lax.fori_loop(0, N_CH, chunk, 0)

    def drain(j, carry):
        pltpu.make_async_copy(ones_v, cnt_sh.at[dst_v.at[0]], sem).wait()
        return carry

    lax.fori_loop(0, N_CH, drain, 0)
    plsc.subcore_barrier()
    pltpu.sync_copy(cnt_sh.at[pl.ds(rs, ROWS_PT)],
                    out_hbm.at[c].at[pl.ds(rs, ROWS_PT)])


@functools.lru_cache(maxsize=None)
def _counts_sc():
    return pl.kernel(
        _counts_body,
        mesh=plsc.VectorSubcoreMesh(core_axis_name="c", subcore_axis_name="s"),
        out_type=jax.ShapeDtypeStruct((NC, N_PAD, CW), jnp.float32),
        scratch_types=[
            pltpu.VMEM((N_CH, CH), jnp.int32),
            pltpu.VMEM((CH, CW), jnp.float32),
            pltpu.VMEM_SHARED((N_PAD, CW), jnp.float32),
            pltpu.SemaphoreType.DMA,
        ],
    )


def _tc_layer_body(h_ref, p_ref, c_ref, w1_ref, b1_ref, g_ref, be_ref,
                   w2_ref, b2_ref, out_ref):
    h = h_ref[...]
    ssum = p_ref[0, :N, :] + p_ref[1, :N, :] + h
    cnt = c_ref[0, :N, 0:1] + c_ref[1, :N, 0:1] + 1.0
    z = h + ssum / cnt
    z1 = jnp.dot(z, w1_ref[...], preferred_element_type=jnp.float32) + b1_ref[...]
    mu = jnp.mean(z1, axis=0, keepdims=True)
    var = jnp.mean((z1 - mu) * (z1 - mu), axis=0, keepdims=True)
    zn = (z1 - mu) * lax.rsqrt(var + 1e-5) * g_ref[...] + be_ref[...]
    zn = jnp.maximum(zn, 0.0)
    out_ref[...] = (jnp.dot(zn, w2_ref[...],
                            preferred_element_type=jnp.float32) + b2_ref[...])


_tc_layer = pl.pallas_call(
    _tc_layer_body,
    out_shape=jax.ShapeDtypeStruct((N, H), jnp.float32),
)


def _tc_head_body(h1_ref, h2_ref, h3_ref, h4_ref, bt_ref,
                  a0n_ref, a0g_ref, b0_ref, bb0_ref, c0_ref,
                  pa_ref, pb_ref, pbb_ref, pc_ref, out_ref):
    f32 = jnp.float32
    npool = h1_ref[...] + h2_ref[...] + h3_ref[...] + h4_ref[...]
    bt = bt_ref[...]
    iota_b = lax.broadcasted_iota(jnp.int32, (B, N), 0)
    eq = (bt == iota_b).astype(f32)
    bcnt = jnp.sum(eq, axis=1, keepdims=True)
    mnorm = eq / jnp.maximum(bcnt, 1.0)
    gp = jnp.dot(mnorm, npool, preferred_element_type=f32)
    row = lax.broadcasted_iota(jnp.int32, (N, B), 0)
    col = lax.broadcasted_iota(jnp.int32, (N, B), 1) * (N // B)
    rep = (jnp.logical_and(row >= col, row < col + (N // B))).astype(f32)
    grph = jnp.dot(rep, gp, preferred_element_type=f32)
    z = jnp.tanh(jnp.dot(npool, a0n_ref[...], preferred_element_type=f32)
                 + jnp.dot(grph, a0g_ref[...], preferred_element_type=f32)
                 + b0_ref[...])
    z = jnp.dot(z, bb0_ref[...], preferred_element_type=f32) + c0_ref[...]
    for l in range(POLICY_L - 1):
        z = jnp.tanh(jnp.dot(z, pa_ref[l], preferred_element_type=f32)
                     + pb_ref[l:l + 1, :])
        z = jnp.dot(z, pbb_ref[l], preferred_element_type=f32) + pc_ref[l:l + 1, :]
    z3 = z.reshape(B, N // B, H)
    out_ref[...] = lax.dot_general(
        z3, z3, (((2,), (2,)), ((0,), (0,))), preferred_element_type=f32)


_tc_head = pl.pallas_call(
    _tc_head_body,
    out_shape=jax.ShapeDtypeStruct((B, N // B, N // B), jnp.float32),
)


def kernel(x, edge_index, batch, gin_W1, gin_b1, gin_gamma, gin_beta,
           gin_W2, gin_b2, pA0, pb0, pB0, pc0, pA, pb, pB, pc):
    src = edge_index[0]
    dst = edge_index[1]
    spad = SE_PAD - E
    src_f = jnp.concatenate([src, jnp.zeros((spad,), jnp.int32)])
    dst_f = jnp.concatenate([dst, jnp.full((spad,), N, jnp.int32)])
    pad = E_PAD - E
    dst_p = jnp.concatenate(
        [dst, jnp.full((pad,), N, jnp.int32)]).reshape(NW, N_CH, CH)
    zrow = jnp.zeros((N_PAD, D), jnp.float32)
    zcnt = jnp.zeros((N_PAD, CW), jnp.float32)
    ones = jnp.ones((CH, CW), jnp.float32)

    cnts = _counts_sc()(dst_p, zcnt, ones)
    hs = []
    h = x
    for l in range(L):
        parts = _seg_sum_sc()(h, src_f, dst_f, zrow)
        h = _tc_layer(h, parts, cnts, gin_W1[l],
                      gin_b1[l].reshape(1, H), gin_gamma[l].reshape(1, H),
                      gin_beta[l].reshape(1, H), gin_W2[l],
                      gin_b2[l].reshape(1, H))
        hs.append(h)

    return _tc_head(hs[0], hs[1], hs[2], hs[3],
                    batch.reshape(1, N).astype(jnp.int32),
                    pA0[:H], pA0[H:], pb0.reshape(1, H), pB0,
                    pc0.reshape(1, H), pA, pb, pB, pc)
